# baseline (speedup 1.0000x reference)
import jax
import jax.numpy as jnp
from jax.experimental import pallas as pl
from jax.experimental.pallas import tpu as pltpu

B = 16384
INPUT_DIM = 128
EMBED_DIM = 16
BLK = 4096
GRID = B // BLK
SUB = 512
NSUB = BLK // SUB


def _fused_kernel(enc_ref, floors_ref, emb_ref, w_ref, b_ref, out_ref):
    w2 = w_ref[:, INPUT_DIM:]
    p = jax.lax.dot_general(
        emb_ref[...], w2,
        dimension_numbers=(((1,), (1,)), ((), ())),
        preferred_element_type=jnp.float32,
    )
    pdiff = p[1:2, :] - p[0:1, :]
    base = p[0:1, :] + b_ref[...][None, :]
    w1 = w_ref[:, :INPUT_DIM]
    for s in range(NSUB):
        rows = pl.ds(s * SUB, SUB)
        dense = jax.lax.dot_general(
            enc_ref[rows, :], w1,
            dimension_numbers=(((1,), (1,)), ((), ())),
            preferred_element_type=jnp.float32,
        )
        f = floors_ref[rows].astype(jnp.float32)[:, None]
        out_ref[rows, :] = (dense + base) + f * pdiff


def kernel(encodings, src_floors, emb_table, W, b):
    return pl.pallas_call(
        _fused_kernel,
        grid=(GRID,),
        in_specs=[
            pl.BlockSpec((BLK, INPUT_DIM), lambda i: (i, 0)),
            pl.BlockSpec((BLK,), lambda i: (i,)),
            pl.BlockSpec((2, EMBED_DIM), lambda i: (0, 0)),
            pl.BlockSpec((INPUT_DIM, INPUT_DIM + EMBED_DIM), lambda i: (0, 0)),
            pl.BlockSpec((INPUT_DIM,), lambda i: (0,)),
        ],
        out_specs=pl.BlockSpec((BLK, INPUT_DIM), lambda i: (i, 0)),
        out_shape=jax.ShapeDtypeStruct((B, INPUT_DIM), jnp.float32),
        compiler_params=pltpu.CompilerParams(
            dimension_semantics=("arbitrary",),
        ),
    )(encodings, src_floors, emb_table, W, b)


# speedup vs baseline: 1.1505x; 1.1505x over previous
import jax
import jax.numpy as jnp
from jax.experimental import pallas as pl
from jax.experimental.pallas import tpu as pltpu

B = 16384
INPUT_DIM = 128
EMBED_DIM = 16
BLK = 8192
GRID = B // BLK
SUB = 512
NSUB = BLK // SUB


def _fused_kernel(enc_ref, floors_ref, emb_ref, w_ref, b_ref, out_ref):
    w2 = w_ref[:, INPUT_DIM:]
    p = jax.lax.dot_general(
        emb_ref[...], w2,
        dimension_numbers=(((1,), (1,)), ((), ())),
        preferred_element_type=jnp.float32,
    )
    pdiff = p[1:2, :] - p[0:1, :]
    base = p[0:1, :] + b_ref[...][None, :]
    w1 = w_ref[:, :INPUT_DIM]
    for s in range(NSUB):
        rows = pl.ds(s * SUB, SUB)
        dense = jax.lax.dot_general(
            enc_ref[rows, :], w1,
            dimension_numbers=(((1,), (1,)), ((), ())),
            preferred_element_type=jnp.float32,
            precision=jax.lax.Precision.DEFAULT,
        )
        f = floors_ref[rows].astype(jnp.float32)[:, None]
        out_ref[rows, :] = (dense + base) + f * pdiff


def kernel(encodings, src_floors, emb_table, W, b):
    return pl.pallas_call(
        _fused_kernel,
        grid=(GRID,),
        in_specs=[
            pl.BlockSpec((BLK, INPUT_DIM), lambda i: (i, 0)),
            pl.BlockSpec((BLK,), lambda i: (i,)),
            pl.BlockSpec((2, EMBED_DIM), lambda i: (0, 0)),
            pl.BlockSpec((INPUT_DIM, INPUT_DIM + EMBED_DIM), lambda i: (0, 0)),
            pl.BlockSpec((INPUT_DIM,), lambda i: (0,)),
        ],
        out_specs=pl.BlockSpec((BLK, INPUT_DIM), lambda i: (i, 0)),
        out_shape=jax.ShapeDtypeStruct((B, INPUT_DIM), jnp.float32),
        compiler_params=pltpu.CompilerParams(
            dimension_semantics=("arbitrary",),
        ),
    )(encodings, src_floors, emb_table, W, b)


# speedup vs baseline: 1.1528x; 1.0020x over previous
import jax
import jax.numpy as jnp
from jax.experimental import pallas as pl
from jax.experimental.pallas import tpu as pltpu

B = 16384
INPUT_DIM = 128
EMBED_DIM = 16
BLK = 8192
GRID = B // BLK
SUB = 512
NSUB = BLK // SUB


def _fused_kernel(enc_ref, floors_ref, emb_ref, w_ref, b_ref, out_ref):
    w2 = w_ref[:, INPUT_DIM:]
    p = jax.lax.dot_general(
        emb_ref[...], w2,
        dimension_numbers=(((1,), (1,)), ((), ())),
        preferred_element_type=jnp.float32,
    )
    pdiff = p[1:2, :] - p[0:1, :]
    base = p[0:1, :] + b_ref[...][None, :]
    w1 = w_ref[:, :INPUT_DIM].astype(jnp.bfloat16)
    for s in range(NSUB):
        rows = pl.ds(s * SUB, SUB)
        dense = jax.lax.dot_general(
            enc_ref[rows, :].astype(jnp.bfloat16), w1,
            dimension_numbers=(((1,), (1,)), ((), ())),
            preferred_element_type=jnp.float32,
        )
        f = floors_ref[rows].astype(jnp.float32)[:, None]
        out_ref[rows, :] = (dense + base) + f * pdiff


def kernel(encodings, src_floors, emb_table, W, b):
    return pl.pallas_call(
        _fused_kernel,
        grid=(GRID,),
        in_specs=[
            pl.BlockSpec((BLK, INPUT_DIM), lambda i: (i, 0)),
            pl.BlockSpec((BLK,), lambda i: (i,)),
            pl.BlockSpec((2, EMBED_DIM), lambda i: (0, 0)),
            pl.BlockSpec((INPUT_DIM, INPUT_DIM + EMBED_DIM), lambda i: (0, 0)),
            pl.BlockSpec((INPUT_DIM,), lambda i: (0,)),
        ],
        out_specs=pl.BlockSpec((BLK, INPUT_DIM), lambda i: (i, 0)),
        out_shape=jax.ShapeDtypeStruct((B, INPUT_DIM), jnp.float32),
        compiler_params=pltpu.CompilerParams(
            dimension_semantics=("arbitrary",),
        ),
    )(encodings, src_floors, emb_table, W, b)


# speedup vs baseline: 1.1612x; 1.0073x over previous
import jax
import jax.numpy as jnp
from jax.experimental import pallas as pl
from jax.experimental.pallas import tpu as pltpu

B = 16384
INPUT_DIM = 128
EMBED_DIM = 16
BLK = 8192
GRID = B // BLK
SUB = 512
NSUB = BLK // SUB


def _fused_kernel(enc_ref, floors_ref, emb_ref, w_ref, b_ref, out_ref):
    w2 = w_ref[:, INPUT_DIM:]
    p = jax.lax.dot_general(
        emb_ref[...], w2,
        dimension_numbers=(((1,), (1,)), ((), ())),
        preferred_element_type=jnp.float32,
    )
    pdiff = p[1:2, :] - p[0:1, :]
    base = p[0:1, :] + b_ref[...][None, :]
    w1 = w_ref[:, :INPUT_DIM]
    for s in range(NSUB):
        rows = pl.ds(s * SUB, SUB)
        dense = jax.lax.dot_general(
            enc_ref[rows, :], w1,
            dimension_numbers=(((1,), (1,)), ((), ())),
            preferred_element_type=jnp.float32,
        )
        f = floors_ref[rows].astype(jnp.float32)[:, None]
        out_ref[rows, :] = (dense + base) + f * pdiff


def kernel(encodings, src_floors, emb_table, W, b):
    return pl.pallas_call(
        _fused_kernel,
        grid=(GRID,),
        in_specs=[
            pl.BlockSpec((BLK, INPUT_DIM), lambda i: (i, 0)),
            pl.BlockSpec((BLK,), lambda i: (i,)),
            pl.BlockSpec((2, EMBED_DIM), lambda i: (0, 0)),
            pl.BlockSpec((INPUT_DIM, INPUT_DIM + EMBED_DIM), lambda i: (0, 0)),
            pl.BlockSpec((INPUT_DIM,), lambda i: (0,)),
        ],
        out_specs=pl.BlockSpec((BLK, INPUT_DIM), lambda i: (i, 0)),
        out_shape=jax.ShapeDtypeStruct((B, INPUT_DIM), jnp.float32),
        compiler_params=pltpu.CompilerParams(
            dimension_semantics=("arbitrary",),
        ),
    )(encodings, src_floors, emb_table, W, b)
